# Initial kernel scaffold; baseline (speedup 1.0000x reference)
import jax
import jax.numpy as jnp
from jax.experimental import pallas as pl


def kernel(x, edge_index, W1_rel, b1_rel, W1_root, W2_rel, b2_rel, W2_root):
    raise NotImplementedError("write your pallas kernel here")


# speedup vs baseline: 7.2505x; 7.2505x over previous
import functools

import jax
import jax.numpy as jnp
from jax import lax
from jax.experimental import pallas as pl
from jax.experimental.pallas import tpu as pltpu
from jax.experimental.pallas import tpu_sc as plsc

N = 10000
D = 128
E = 320000
NC = 2
NS = 16
NW = NC * NS
EPW = E // NW
CH = 80
NCHUNK = EPW // CH
RPS = 624
RTAIL = N - NS * RPS

_BLK = 2000


def _mm_body(x_ref, w_ref, o_ref):
    o_ref[...] = lax.dot_general(
        x_ref[...], w_ref[...], (((1,), (1,)), ((), ())),
        preferred_element_type=jnp.float32)


def _matmul_t(x, w):
    return pl.pallas_call(
        _mm_body,
        grid=(N // _BLK,),
        in_specs=[pl.BlockSpec((_BLK, D), lambda i: (i, 0)),
                  pl.BlockSpec((D, D), lambda i: (0, 0))],
        out_specs=pl.BlockSpec((_BLK, D), lambda i: (i, 0)),
        out_shape=jax.ShapeDtypeStruct((N, D), jnp.float32),
    )(x, w)


def _mid_body(agg_ref, x_ref, wroot_ref, b_ref, wrel2_ref, h_ref, y2_ref):
    h = (agg_ref[0] + agg_ref[1] + b_ref[...] +
         lax.dot_general(x_ref[...], wroot_ref[...], (((1,), (1,)), ((), ())),
                         preferred_element_type=jnp.float32))
    h = jnp.maximum(h, 0.0)
    h_ref[...] = h
    y2_ref[...] = lax.dot_general(
        h, wrel2_ref[...], (((1,), (1,)), ((), ())),
        preferred_element_type=jnp.float32)


def _mid_stage(aggp, x, wroot, b, wrel2):
    return pl.pallas_call(
        _mid_body,
        grid=(N // _BLK,),
        in_specs=[pl.BlockSpec((2, _BLK, D), lambda i: (0, i, 0)),
                  pl.BlockSpec((_BLK, D), lambda i: (i, 0)),
                  pl.BlockSpec((D, D), lambda i: (0, 0)),
                  pl.BlockSpec((1, D), lambda i: (0, 0)),
                  pl.BlockSpec((D, D), lambda i: (0, 0))],
        out_specs=[pl.BlockSpec((_BLK, D), lambda i: (i, 0)),
                   pl.BlockSpec((_BLK, D), lambda i: (i, 0))],
        out_shape=[jax.ShapeDtypeStruct((N, D), jnp.float32),
                   jax.ShapeDtypeStruct((N, D), jnp.float32)],
    )(aggp, x, wroot, b, wrel2)


def _final_body(agg_ref, h_ref, wroot_ref, b_ref, o_ref):
    o_ref[...] = (agg_ref[0] + agg_ref[1] + b_ref[...] +
                  lax.dot_general(h_ref[...], wroot_ref[...],
                                  (((1,), (1,)), ((), ())),
                                  preferred_element_type=jnp.float32))


def _final_stage(aggp, h, wroot, b):
    return pl.pallas_call(
        _final_body,
        grid=(N // _BLK,),
        in_specs=[pl.BlockSpec((2, _BLK, D), lambda i: (0, i, 0)),
                  pl.BlockSpec((_BLK, D), lambda i: (i, 0)),
                  pl.BlockSpec((D, D), lambda i: (0, 0)),
                  pl.BlockSpec((1, D), lambda i: (0, 0))],
        out_specs=pl.BlockSpec((_BLK, D), lambda i: (i, 0)),
        out_shape=jax.ShapeDtypeStruct((N, D), jnp.float32),
    )(aggp, h, wroot, b)


def _sc_segment_sum(y, src, dst, zeros):
    mesh = plsc.VectorSubcoreMesh(core_axis_name="c", subcore_axis_name="s")

    @functools.partial(
        pl.kernel, mesh=mesh,
        out_type=jax.ShapeDtypeStruct((NC, N, D), jnp.float32),
        scratch_types=[
            pltpu.VMEM((NCHUNK, CH), jnp.int32),
            pltpu.VMEM((NCHUNK, CH), jnp.int32),
            pltpu.VMEM((CH, D), jnp.float32),
            pltpu.VMEM_SHARED((N, D), jnp.float32),
            pltpu.SemaphoreType.DMA,
        ],
    )
    def scat(y_hbm, src_hbm, dst_hbm, zero_hbm, out_hbm,
             src_v, dst_v, rows, acc, sem):
        c = lax.axis_index("c")
        s = lax.axis_index("s")
        wid = s * NC + c
        pltpu.sync_copy(src_hbm.at[wid], src_v)
        pltpu.sync_copy(dst_hbm.at[wid], dst_v)
        pltpu.sync_copy(zero_hbm.at[pl.ds(s * RPS, RPS)],
                        acc.at[pl.ds(s * RPS, RPS)])

        @pl.when(s == 0)
        def _():
            pltpu.sync_copy(zero_hbm.at[pl.ds(NS * RPS, RTAIL)],
                            acc.at[pl.ds(NS * RPS, RTAIL)])

        plsc.subcore_barrier()

        @pl.loop(0, NCHUNK)
        def _(j):
            pltpu.async_copy(y_hbm.at[src_v.at[j]], rows, sem).wait()
            pltpu.sync_copy(rows, acc.at[dst_v.at[j]], add=True)

        plsc.subcore_barrier()
        pltpu.sync_copy(acc.at[pl.ds(s * RPS, RPS)],
                        out_hbm.at[c].at[pl.ds(s * RPS, RPS)])

        @pl.when(s == 0)
        def _():
            pltpu.sync_copy(acc.at[pl.ds(NS * RPS, RTAIL)],
                            out_hbm.at[c].at[pl.ds(NS * RPS, RTAIL)])

    return scat(y, src, dst, zeros)


def kernel(x, edge_index, W1_rel, b1_rel, W1_root, W2_rel, b2_rel, W2_root):
    ei = edge_index.astype(jnp.int32)
    src = ei[0].reshape(NW, NCHUNK, CH)
    dst = ei[1].reshape(NW, NCHUNK, CH)
    zeros = jnp.zeros((N, D), jnp.float32)
    b1 = b1_rel.reshape(1, D)
    b2 = b2_rel.reshape(1, D)

    y1 = _matmul_t(x, W1_rel)
    agg1 = _sc_segment_sum(y1, src, dst, zeros)
    h, y2 = _mid_stage(agg1, x, W1_root, b1, W2_rel)
    agg2 = _sc_segment_sum(y2, src, dst, zeros)
    out = _final_stage(agg2, h, W2_root, b2)
    return out
